# baseline (speedup 1.0000x reference)
import functools

import jax
import jax.numpy as jnp
from jax import lax
from jax.experimental import pallas as pl
from jax.experimental.pallas import tpu as pltpu
from jax.experimental.pallas import tpu_sc as plsc

NC = 2
NS = 16
NW = NC * NS
L = 16

B = 320000
D = 128
E = B // NW
C = 128
NCHUNK = E // C
TAIL = E - NCHUNK * C
K = D // L
NBUF = 3


def _dot_decode_body(z_hbm, src_hbm, dst_hbm, out_hbm,
                     idx_s, idx_d, out_v,
                     rs0, rd0, rs1, rd1, rs2, rd2,
                     sem0, sem1, sem2):
    wid = lax.axis_index("s") * NC + lax.axis_index("c")
    base = pl.multiple_of(wid * E, 8)

    pltpu.sync_copy(src_hbm.at[pl.ds(base, E)], idx_s)
    pltpu.sync_copy(dst_hbm.at[pl.ds(base, E)], idx_d)

    bufs = ((rs0, rd0, sem0), (rs1, rd1, sem1), (rs2, rd2, sem2))

    def issue(c, b, n=C):
        rs, rd, sem = bufs[b]
        h = n // 2
        for j in range(2):
            off = pl.multiple_of(c * C + j * h, 8)
            pltpu.async_copy(z_hbm.at[idx_s.at[pl.ds(off, h)]],
                             rs.at[pl.ds(j * h, h)], sem)
            pltpu.async_copy(z_hbm.at[idx_d.at[pl.ds(off, h)]],
                             rd.at[pl.ds(j * h, h)], sem)

    def wait(c, b, n=C):
        rs, rd, sem = bufs[b]
        h = n // 2
        for j in range(2):
            off = pl.multiple_of(c * C + j * h, 8)
            pltpu.make_async_copy(z_hbm.at[idx_s.at[pl.ds(off, h)]],
                                  rs.at[pl.ds(j * h, h)], sem).wait()
            pltpu.make_async_copy(z_hbm.at[idx_d.at[pl.ds(off, h)]],
                                  rd.at[pl.ds(j * h, h)], sem).wait()

    lane = lax.iota(jnp.int32, L)
    lane15 = lane == (L - 1)

    def edge_body(rs, rd, row, obase):
        prods = [rs[row, pl.ds(k * L, L)] * rd[row, pl.ds(k * L, L)]
                 for k in range(K)]
        while len(prods) > 1:
            prods = [a + b for a, b in zip(prods[0::2], prods[1::2])]
        cum = plsc.cumsum(prods[0])
        plsc.store_compressed(out_v.at[pl.ds(obase, L)], cum, mask=lane15)

    def compute(c, b, n=C):
        rs, rd, _ = bufs[b]

        @plsc.parallel_loop(0, n, unroll=4)
        def _(e):
            edge_body(rs, rd, e, c * C + e)

    for b in range(NBUF):
        issue(b, b)

    def outer(i, _):
        for b in range(NBUF):
            c = NBUF * i + b
            wait(c, b)

            @pl.when(c + NBUF < NCHUNK)
            def _():
                issue(c + NBUF, b)
        return 0

    lax.fori_loop(0, NCHUNK // NBUF, outer, 0)

    issue(NCHUNK, 0, n=TAIL)
    wait(NCHUNK, 0, n=TAIL)
    compute(NCHUNK, 0, n=TAIL)

    def sig_body(g, _):
        off = pl.multiple_of(g * L, 8)
        v = out_v[pl.ds(off, L)]
        out_v[pl.ds(off, L)] = 1.0 / (1.0 + jnp.exp(-v))
        return 0

    lax.fori_loop(0, E // L, sig_body, 0, unroll=2)
    pltpu.sync_copy(out_v.at[pl.ds(0, E)], out_hbm.at[pl.ds(base, E)])


@jax.jit
def kernel(z, edge_index):
    mesh = plsc.VectorSubcoreMesh(core_axis_name="c", subcore_axis_name="s")
    f = pl.kernel(
        _dot_decode_body,
        out_type=jax.ShapeDtypeStruct((B,), jnp.float32),
        mesh=mesh,
        compiler_params=pltpu.CompilerParams(needs_layout_passes=False),
        scratch_types=[
            pltpu.VMEM((E,), jnp.int32),
            pltpu.VMEM((E,), jnp.int32),
            pltpu.VMEM((E + L,), jnp.float32),
            *([pltpu.VMEM((C, D), jnp.float32)] * (2 * NBUF)),
            *([pltpu.SemaphoreType.DMA] * NBUF),
        ],
    )
    return f(z, edge_index[0], edge_index[1])


# speedup vs baseline: 1.0634x; 1.0634x over previous
import functools

import jax
import jax.numpy as jnp
from jax import lax
from jax.experimental import pallas as pl
from jax.experimental.pallas import tpu as pltpu
from jax.experimental.pallas import tpu_sc as plsc

NC = 2
NS = 16
NW = NC * NS
L = 16

B = 320000
D = 128
E = B // NW
C = 128
NCHUNK = E // C
TAIL = E - NCHUNK * C
K = D // L
NBUF = 3


def _dot_decode_body(z_hbm, src_hbm, dst_hbm, out_hbm,
                     idx_s, idx_d, out_v,
                     rs0, rd0, rs1, rd1, rs2, rd2,
                     sem0, sem1, sem2):
    wid = lax.axis_index("s") * NC + lax.axis_index("c")
    base = pl.multiple_of(wid * E, 8)

    pltpu.sync_copy(src_hbm.at[pl.ds(base, E)], idx_s)
    pltpu.sync_copy(dst_hbm.at[pl.ds(base, E)], idx_d)

    bufs = ((rs0, rd0, sem0), (rs1, rd1, sem1), (rs2, rd2, sem2))

    def issue(c, b, n=C):
        rs, rd, sem = bufs[b]
        h = n // 2
        for j in range(2):
            off = pl.multiple_of(c * C + j * h, 8)
            pltpu.async_copy(z_hbm.at[idx_s.at[pl.ds(off, h)]],
                             rs.at[pl.ds(j * h, h)], sem)
            pltpu.async_copy(z_hbm.at[idx_d.at[pl.ds(off, h)]],
                             rd.at[pl.ds(j * h, h)], sem)

    def wait(c, b, n=C):
        rs, rd, sem = bufs[b]
        h = n // 2
        for j in range(2):
            off = pl.multiple_of(c * C + j * h, 8)
            pltpu.make_async_copy(z_hbm.at[idx_s.at[pl.ds(off, h)]],
                                  rs.at[pl.ds(j * h, h)], sem).wait()
            pltpu.make_async_copy(z_hbm.at[idx_d.at[pl.ds(off, h)]],
                                  rd.at[pl.ds(j * h, h)], sem).wait()

    lane = lax.iota(jnp.int32, L)
    lane15 = lane == (L - 1)

    def edge_body(rs, rd, row, obase):
        prods = [rs[row, pl.ds(k * L, L)] * rd[row, pl.ds(k * L, L)]
                 for k in range(K)]
        while len(prods) > 1:
            prods = [a + b for a, b in zip(prods[0::2], prods[1::2])]
        cum = plsc.cumsum(prods[0])
        sig = 1.0 / (1.0 + jnp.exp(-cum))
        plsc.store_compressed(out_v.at[pl.ds(obase, L)], sig, mask=lane15)

    def compute(c, b, n=C):
        rs, rd, _ = bufs[b]

        @plsc.parallel_loop(0, n, unroll=4)
        def _(e):
            edge_body(rs, rd, e, c * C + e)

    for b in range(NBUF):
        issue(b, b)

    def outer(i, _):
        for b in range(NBUF):
            c = NBUF * i + b
            wait(c, b)
            compute(c, b)

            @pl.when(c + NBUF < NCHUNK)
            def _():
                issue(c + NBUF, b)
        return 0

    lax.fori_loop(0, NCHUNK // NBUF, outer, 0)

    issue(NCHUNK, 0, n=TAIL)
    wait(NCHUNK, 0, n=TAIL)
    compute(NCHUNK, 0, n=TAIL)

    pltpu.sync_copy(out_v.at[pl.ds(0, E)], out_hbm.at[pl.ds(base, E)])


@jax.jit
def kernel(z, edge_index):
    mesh = plsc.VectorSubcoreMesh(core_axis_name="c", subcore_axis_name="s")
    f = pl.kernel(
        _dot_decode_body,
        out_type=jax.ShapeDtypeStruct((B,), jnp.float32),
        mesh=mesh,
        compiler_params=pltpu.CompilerParams(needs_layout_passes=False),
        scratch_types=[
            pltpu.VMEM((E,), jnp.int32),
            pltpu.VMEM((E,), jnp.int32),
            pltpu.VMEM((E + L,), jnp.float32),
            *([pltpu.VMEM((C, D), jnp.float32)] * (2 * NBUF)),
            *([pltpu.SemaphoreType.DMA] * NBUF),
        ],
    )
    return f(z, edge_index[0], edge_index[1])
